# baseline (speedup 1.0000x reference)
import jax
import jax.numpy as jnp
from jax.experimental import pallas as pl
from jax.experimental.pallas import tpu as pltpu
from jax.experimental.pallas import tpu_sc as plsc

_UNITS = 32
_L = 16384


def kernel(x, pe, position_ids):
    batch, seq_len, d_model = x.shape
    n_pe = seq_len * d_model
    xf = x.reshape(batch * n_pe)
    pef = pe[:seq_len].reshape(n_pe)

    pe_blocks = n_pe // _L
    chunks_per_unit = pe_blocks // _UNITS
    inner = chunks_per_unit * batch

    def x_index(i, j):
        return (i * inner + j,)

    def pe_index(i, j):
        return ((i * inner + j) % pe_blocks,)

    mesh = plsc.VectorSubcoreMesh(
        core_axis_name="core", subcore_axis_name="subcore"
    )

    @pl.kernel(out_type=jax.ShapeDtypeStruct(xf.shape, x.dtype), mesh=mesh)
    def sc_add(x_hbm, pe_hbm, o_hbm):
        def body(x_vmem, pe_vmem, o_vmem):
            @pl.loop(0, _L, step=16)
            def _(i):
                sl = pl.ds(i, 16)
                o_vmem[sl] = x_vmem[sl] + pe_vmem[sl]

        pltpu.emit_pipeline(
            body,
            grid=(_UNITS, inner),
            in_specs=[
                pl.BlockSpec((_L,), index_map=x_index),
                pl.BlockSpec((_L,), index_map=pe_index),
            ],
            out_specs=[pl.BlockSpec((_L,), index_map=x_index)],
            core_axis_name=("core", "subcore"),
            dimension_semantics=(pltpu.PARALLEL, pltpu.ARBITRARY),
        )(x_hbm, pe_hbm, o_hbm)

    return sc_add(xf, pef).reshape(x.shape)


# speedup vs baseline: 1.0613x; 1.0613x over previous
import jax
import jax.numpy as jnp
from jax.experimental import pallas as pl
from jax.experimental.pallas import tpu as pltpu
from jax.experimental.pallas import tpu_sc as plsc

_UNITS = 32
_L = 16384


def kernel(x, pe, position_ids):
    batch, seq_len, d_model = x.shape
    n_pe = seq_len * d_model
    xf = x.reshape(batch * n_pe)
    pef = pe[:seq_len].reshape(n_pe)

    pe_blocks = n_pe // _L
    chunks_per_unit = pe_blocks // _UNITS
    inner = chunks_per_unit * batch

    def x_index(i, j):
        return (i * inner + j,)

    def pe_index(i, j):
        return ((i * inner + j) % pe_blocks,)

    mesh = plsc.VectorSubcoreMesh(
        core_axis_name="core", subcore_axis_name="subcore"
    )

    @pl.kernel(out_type=jax.ShapeDtypeStruct(xf.shape, x.dtype), mesh=mesh)
    def sc_add(x_hbm, pe_hbm, o_hbm):
        def body(x_vmem, pe_vmem, o_vmem):
            @pl.loop(0, _L, step=256)
            def _(i):
                for k in range(16):
                    sl = pl.ds(i + k * 16, 16)
                    o_vmem[sl] = x_vmem[sl] + pe_vmem[sl]

        pltpu.emit_pipeline(
            body,
            grid=(_UNITS, inner),
            in_specs=[
                pl.BlockSpec((_L,), index_map=x_index),
                pl.BlockSpec((_L,), index_map=pe_index),
            ],
            out_specs=[pl.BlockSpec((_L,), index_map=x_index)],
            core_axis_name=("core", "subcore"),
            dimension_semantics=(pltpu.PARALLEL, pltpu.ARBITRARY),
        )(x_hbm, pe_hbm, o_hbm)

    return sc_add(xf, pef).reshape(x.shape)


# speedup vs baseline: 1.1119x; 1.0476x over previous
import jax
import jax.numpy as jnp
from jax.experimental import pallas as pl
from jax.experimental.pallas import tpu as pltpu
from jax.experimental.pallas import tpu_sc as plsc

_UNITS = 32
_L = 4096


def kernel(x, pe, position_ids):
    batch, seq_len, d_model = x.shape
    n_pe = seq_len * d_model
    xf = x.reshape(batch * n_pe)
    pef = pe[:seq_len].reshape(n_pe)

    pe_blocks = n_pe // _L
    chunks_per_unit = pe_blocks // _UNITS
    inner = chunks_per_unit * batch

    def x_index(i, j):
        return (i * inner + j,)

    def pe_index(i, j):
        return ((i * inner + j) % pe_blocks,)

    mesh = plsc.VectorSubcoreMesh(
        core_axis_name="core", subcore_axis_name="subcore"
    )

    @pl.kernel(out_type=jax.ShapeDtypeStruct(xf.shape, x.dtype), mesh=mesh)
    def sc_add(x_hbm, pe_hbm, o_hbm):
        def body(x_vmem, pe_vmem, o_vmem):
            for k in range(_L // 16):
                sl = slice(k * 16, (k + 1) * 16)
                o_vmem[sl] = x_vmem[sl] + pe_vmem[sl]

        pltpu.emit_pipeline(
            body,
            grid=(_UNITS, inner),
            in_specs=[
                pl.BlockSpec((_L,), index_map=x_index),
                pl.BlockSpec((_L,), index_map=pe_index),
            ],
            out_specs=[pl.BlockSpec((_L,), index_map=x_index)],
            core_axis_name=("core", "subcore"),
            dimension_semantics=(pltpu.PARALLEL, pltpu.ARBITRARY),
        )(x_hbm, pe_hbm, o_hbm)

    return sc_add(xf, pef).reshape(x.shape)


# speedup vs baseline: 1.4937x; 1.3434x over previous
import jax
import jax.numpy as jnp
from jax.experimental import pallas as pl
from jax.experimental.pallas import tpu as pltpu
from jax.experimental.pallas import tpu_sc as plsc

_UNITS = 32
_L = 4096


def kernel(x, pe, position_ids):
    batch, seq_len, d_model = x.shape
    n_pe = seq_len * d_model
    xf = x.reshape(batch * n_pe)
    pef = pe[:seq_len].reshape(n_pe)

    pe_blocks = n_pe // _L
    chunks_per_unit = pe_blocks // _UNITS
    inner = chunks_per_unit * batch

    def x_index(i, j):
        return (i * inner + j,)

    def pe_index(i, j):
        return ((i * inner + j) % pe_blocks,)

    mesh = plsc.VectorSubcoreMesh(
        core_axis_name="core", subcore_axis_name="subcore"
    )

    @pl.kernel(out_type=jax.ShapeDtypeStruct(xf.shape, x.dtype), mesh=mesh)
    def sc_add(x_hbm, pe_hbm, o_hbm):
        def body(x_vmem, pe_vmem, o_vmem):
            @plsc.parallel_loop(0, _L, step=16, unroll=4)
            def _(i):
                sl = pl.ds(i, 16)
                o_vmem[sl] = x_vmem[sl] + pe_vmem[sl]

        pltpu.emit_pipeline(
            body,
            grid=(_UNITS, inner),
            in_specs=[
                pl.BlockSpec((_L,), index_map=x_index),
                pl.BlockSpec((_L,), index_map=pe_index),
            ],
            out_specs=[pl.BlockSpec((_L,), index_map=x_index)],
            core_axis_name=("core", "subcore"),
            dimension_semantics=(pltpu.PARALLEL, pltpu.ARBITRARY),
        )(x_hbm, pe_hbm, o_hbm)

    return sc_add(xf, pef).reshape(x.shape)


# speedup vs baseline: 7.4393x; 4.9803x over previous
import jax
import jax.numpy as jnp
from jax.experimental import pallas as pl


def _add_pe_block(x_ref, pe_ref, o_ref):
    o_ref[...] = x_ref[...] + pe_ref[...][None, :, :]


def kernel(x, pe, position_ids):
    batch, seq_len, d_model = x.shape
    blk = 256
    grid = (seq_len // blk,)
    return pl.pallas_call(
        _add_pe_block,
        grid=grid,
        in_specs=[
            pl.BlockSpec((batch, blk, d_model), lambda i: (0, i, 0)),
            pl.BlockSpec((blk, d_model), lambda i: (i, 0)),
        ],
        out_specs=pl.BlockSpec((batch, blk, d_model), lambda i: (0, i, 0)),
        out_shape=jax.ShapeDtypeStruct(x.shape, x.dtype),
    )(x, pe[:seq_len])
